# baseline (speedup 1.0000x reference)
import jax
import jax.numpy as jnp
from jax.experimental import pallas as pl
from jax.experimental.pallas import tpu as pltpu
from jax.experimental.pallas import tpu_sc as plsc

_EMBED = 64
_SCALE = 8.0
_WINDOW = 128
_LANES = 16


def kernel(inputTensor, table):
    batch, seq = inputTensor.shape
    num_idx = batch * seq
    idx = inputTensor.reshape(1, num_idx)

    mesh = plsc.VectorSubcoreMesh(
        core_axis_name="core", subcore_axis_name="subcore"
    )

    @jax.jit
    @pl.kernel(
        out_type=jax.ShapeDtypeStruct((num_idx, _EMBED), table.dtype),
        mesh=mesh,
        compiler_params=pltpu.CompilerParams(use_tc_tiling_on_sc=False),
    )
    def gather_scale(table_hbm, idx_hbm, out_hbm):
        def body(i_vmem, o_vmem):
            pltpu.sync_copy(table_hbm.at[i_vmem.at[0]], o_vmem)

            @pl.loop(0, _WINDOW)
            def _(r):
                @pl.loop(0, _EMBED, step=_LANES)
                def _(c):
                    slc = (pl.ds(r, 1), pl.ds(c, _LANES))
                    o_vmem.at[*slc][...] = o_vmem.at[*slc][...] * _SCALE

        pltpu.emit_pipeline(
            body,
            grid=(num_idx // _WINDOW,),
            in_specs=[
                pl.BlockSpec((1, _WINDOW), index_map=lambda i: (0, i))
            ],
            out_specs=[
                pl.BlockSpec((_WINDOW, _EMBED), index_map=lambda i: (i, 0))
            ],
            core_axis_name=("core", "subcore"),
            dimension_semantics=(pltpu.PARALLEL,),
        )(idx_hbm, out_hbm)

    out = gather_scale(table, idx)
    return out.reshape(batch, seq, _EMBED)


# speedup vs baseline: 1.0046x; 1.0046x over previous
import jax
import jax.numpy as jnp
from jax import lax
from jax.experimental import pallas as pl
from jax.experimental.pallas import tpu as pltpu
from jax.experimental.pallas import tpu_sc as plsc

_EMBED = 64
_SCALE = 8.0
_WINDOW = 128
_LANES = 16


def kernel(inputTensor, table):
    batch, seq = inputTensor.shape
    num_idx = batch * seq
    idx = inputTensor.reshape(1, num_idx)

    mesh = plsc.VectorSubcoreMesh(
        core_axis_name="core", subcore_axis_name="subcore"
    )

    @jax.jit
    @pl.kernel(
        out_type=jax.ShapeDtypeStruct((num_idx, _EMBED), table.dtype),
        mesh=mesh,
        compiler_params=pltpu.CompilerParams(use_tc_tiling_on_sc=False),
    )
    def gather_scale(table_hbm, idx_hbm, out_hbm):
        def body(i_vmem, o_vmem):
            pltpu.sync_copy(table_hbm.at[i_vmem.at[0]], o_vmem)

            def scale_row(r, carry):
                for c in range(_EMBED // _LANES):
                    sl = pl.ds(c * _LANES, _LANES)
                    o_vmem[r, sl] = o_vmem[r, sl] * _SCALE
                return carry

            lax.fori_loop(0, _WINDOW, scale_row, 0)

        pltpu.emit_pipeline(
            body,
            grid=(num_idx // _WINDOW,),
            in_specs=[
                pl.BlockSpec((1, _WINDOW), index_map=lambda i: (0, i))
            ],
            out_specs=[
                pl.BlockSpec((_WINDOW, _EMBED), index_map=lambda i: (i, 0))
            ],
            core_axis_name=("core", "subcore"),
            dimension_semantics=(pltpu.PARALLEL,),
        )(idx_hbm, out_hbm)

    out = gather_scale(table, idx)
    return out.reshape(batch, seq, _EMBED)


# speedup vs baseline: 1.4994x; 1.4925x over previous
import jax
import jax.numpy as jnp
from jax import lax
from jax.experimental import pallas as pl
from jax.experimental.pallas import tpu as pltpu
from jax.experimental.pallas import tpu_sc as plsc

_EMBED = 64
_SCALE = 8.0
_GATHER = 128
_NBUF = 4
_LANES = 16


def kernel(inputTensor, table):
    batch, seq = inputTensor.shape
    num_idx = batch * seq
    idx = inputTensor.reshape(num_idx)

    info = plsc.get_sparse_core_info()
    n_workers = info.num_cores * info.num_subcores
    rows_pw = num_idx // n_workers
    turns = rows_pw // (_GATHER * _NBUF)

    mesh = plsc.VectorSubcoreMesh(
        core_axis_name="core", subcore_axis_name="subcore"
    )

    @jax.jit
    @pl.kernel(
        out_type=jax.ShapeDtypeStruct((num_idx, _EMBED), table.dtype),
        mesh=mesh,
        scratch_types=[
            pltpu.VMEM((rows_pw,), jnp.int32),
            pltpu.VMEM((_NBUF, _GATHER, _EMBED), jnp.float32),
            pltpu.VMEM((_NBUF, _GATHER, _EMBED), jnp.float32),
            pltpu.SemaphoreType.DMA((_NBUF,)),
            pltpu.SemaphoreType.DMA((_NBUF,)),
        ],
        compiler_params=pltpu.CompilerParams(use_tc_tiling_on_sc=False),
    )
    def gather_scale(table_hbm, idx_hbm, out_hbm, idx_v, gbuf, sbuf, gsem, osem):
        wid = lax.axis_index("subcore") * info.num_cores + lax.axis_index("core")
        base = wid * rows_pw
        pltpu.sync_copy(idx_hbm.at[pl.ds(base, rows_pw)], idx_v)

        def start_gather(i, b):
            pltpu.async_copy(
                table_hbm.at[idx_v.at[pl.ds(i * _GATHER, _GATHER)]],
                gbuf.at[b],
                gsem.at[b],
            )

        def wait_gather(b):
            pltpu.make_async_copy(
                table_hbm.at[pl.ds(0, _GATHER)], gbuf.at[b], gsem.at[b]
            ).wait()

        def start_out(i, b):
            pltpu.async_copy(
                sbuf.at[b],
                out_hbm.at[pl.ds(base + i * _GATHER, _GATHER)],
                osem.at[b],
            )

        def wait_out(b):
            pltpu.make_async_copy(
                sbuf.at[b], out_hbm.at[pl.ds(base, _GATHER)], osem.at[b]
            ).wait()

        def scale(b):
            src = gbuf.at[b]
            dst = sbuf.at[b]

            def row(r, carry):
                for c in range(_EMBED // _LANES):
                    sl = pl.ds(c * _LANES, _LANES)
                    dst[r, sl] = src[r, sl] * _SCALE
                return carry

            lax.fori_loop(0, _GATHER, row, 0)

        for b in range(_NBUF):
            start_gather(b, b)

        def turn(j, carry):
            for b in range(_NBUF):
                i = j * _NBUF + b
                wait_gather(b)

                @pl.when(j > 0)
                def _wait_prev_out():
                    wait_out(b)

                scale(b)

                @pl.when(j < turns - 1)
                def _start_next_gather():
                    start_gather(i + _NBUF, b)

                start_out(i, b)
            return carry

        lax.fori_loop(0, turns, turn, 0)

        for b in range(_NBUF):
            wait_out(b)

    out = gather_scale(table, idx)
    return out.reshape(batch, seq, _EMBED)
